# baseline (speedup 1.0000x reference)
import functools

import jax
import jax.numpy as jnp
from jax import lax
from jax.experimental import pallas as pl
from jax.experimental.pallas import tpu as pltpu
from jax.experimental.pallas import tpu_sc as plsc


def _gather_kernel(nw, per_w, nchunks, chunk, d, n):
    mesh = plsc.VectorSubcoreMesh(core_axis_name="c", subcore_axis_name="s")

    @functools.partial(
        pl.kernel,
        mesh=mesh,
        out_type=jax.ShapeDtypeStruct((n, d), jnp.float32),
        compiler_params=pltpu.CompilerParams(use_tc_tiling_on_sc=False),
        scratch_types=[
            pltpu.VMEM((nchunks, chunk), jnp.int32),
            pltpu.VMEM((chunk, d), jnp.float32),
            pltpu.SemaphoreType.DMA,
        ],
    )
    def k(x_hbm, table_hbm, out_hbm, idx_v, rows_v, sem):
        wid = lax.axis_index("s") * 2 + lax.axis_index("c")
        base = wid * per_w
        pltpu.sync_copy(x_hbm.at[wid], idx_v)

        def body(j, carry):
            pltpu.async_copy(table_hbm.at[idx_v.at[j]], rows_v, sem).wait()
            pltpu.sync_copy(rows_v, out_hbm.at[pl.ds(base + j * chunk, chunk)])
            return carry

        lax.fori_loop(0, nchunks, body, 0)

    return k


def kernel(x, table):
    b, h = x.shape
    v, d = table.shape
    n = b * h
    nw = 32
    per_w = n // nw
    chunk = 128
    nchunks = per_w // chunk
    xr = x.reshape(nw, nchunks, chunk).astype(jnp.int32)
    out = _gather_kernel(nw, per_w, nchunks, chunk, d, n)(xr, table)
    return out.reshape(b, h, d)


# speedup vs baseline: 1.1095x; 1.1095x over previous
import functools

import jax
import jax.numpy as jnp
from jax import lax
from jax.experimental import pallas as pl
from jax.experimental.pallas import tpu as pltpu
from jax.experimental.pallas import tpu_sc as plsc

_NBUF = 4


def _gather_kernel(nw, per_w, nchunks, chunk, d, n):
    mesh = plsc.VectorSubcoreMesh(core_axis_name="c", subcore_axis_name="s")
    ngroups = nchunks // _NBUF

    @functools.partial(
        pl.kernel,
        mesh=mesh,
        out_type=jax.ShapeDtypeStruct((n, d), jnp.float32),
        compiler_params=pltpu.CompilerParams(use_tc_tiling_on_sc=False),
        scratch_types=[
            pltpu.VMEM((nchunks, chunk), jnp.int32),
            *[pltpu.VMEM((chunk, d), jnp.float32) for _ in range(_NBUF)],
            *[pltpu.SemaphoreType.DMA for _ in range(2 * _NBUF)],
        ],
    )
    def k(x_hbm, table_hbm, out_hbm, idx_v, *bufs_and_sems):
        bufs = bufs_and_sems[:_NBUF]
        gsem = bufs_and_sems[_NBUF:2 * _NBUF]
        ssem = bufs_and_sems[2 * _NBUF:]
        wid = lax.axis_index("s") * 2 + lax.axis_index("c")
        base = wid * per_w
        pltpu.sync_copy(x_hbm.at[wid], idx_v)

        def start_gather(j, b):
            pltpu.async_copy(table_hbm.at[idx_v.at[j]], bufs[b], gsem[b])

        def wait_gather(j, b):
            pltpu.make_async_copy(
                table_hbm.at[idx_v.at[j]], bufs[b], gsem[b]).wait()

        def start_scatter(j, b):
            pltpu.async_copy(
                bufs[b], out_hbm.at[pl.ds(base + j * chunk, chunk)], ssem[b])

        def wait_scatter(j, b):
            pltpu.make_async_copy(
                bufs[b], out_hbm.at[pl.ds(base + j * chunk, chunk)],
                ssem[b]).wait()

        for b in range(_NBUF):
            start_gather(b, b)

        def body(g, carry):
            for b in range(_NBUF):
                j = g * _NBUF + b
                wait_gather(j, b)
                start_scatter(j, b)
            for b in range(_NBUF):
                j = g * _NBUF + b
                wait_scatter(j, b)
                start_gather(j + _NBUF, b)
            return carry

        lax.fori_loop(0, ngroups - 1, body, 0)

        last = (ngroups - 1) * _NBUF
        for b in range(_NBUF):
            wait_gather(last + b, b)
            start_scatter(last + b, b)
        for b in range(_NBUF):
            wait_scatter(last + b, b)

    return k


def kernel(x, table):
    b, h = x.shape
    v, d = table.shape
    n = b * h
    nw = 32
    per_w = n // nw
    chunk = 128
    nchunks = per_w // chunk
    xr = x.reshape(nw, nchunks, chunk).astype(jnp.int32)
    out = _gather_kernel(nw, per_w, nchunks, chunk, d, n)(xr, table)
    return out.reshape(b, h, d)
